# initial kernel scaffold (unmeasured)
import jax
import jax.numpy as jnp
from jax import lax
from jax.experimental import pallas as pl
from jax.experimental.pallas import tpu as pltpu


def kernel(Q, K, V):
    b, q_len, h, d = Q.shape
    k_len = K.shape[1]
    hd = h * d
    scale = d ** -0.5

    Qf = Q.reshape(b, hd)
    Kf = K.reshape(b, k_len, hd)
    Vf = V.reshape(b, k_len, hd)

    def body(q_ref, k_ref, v_ref, out_ref, acc_ref, recv_ref, send_sem, recv_sem):
        bi = pl.program_id(0)
        mx = lax.axis_index("x")
        my = lax.axis_index("y")
        mz = lax.axis_index("z")
        partner = (mx, my, 1 - mz)

        @pl.when(bi == 0)
        def _entry_barrier():
            bar = pltpu.get_barrier_semaphore()
            pl.semaphore_signal(
                bar, inc=1, device_id=partner,
                device_id_type=pl.DeviceIdType.MESH,
            )
            pl.semaphore_wait(bar, 1)

        lane = lax.broadcasted_iota(jnp.int32, (h, hd), 1)
        sub = lax.broadcasted_iota(jnp.int32, (h, hd), 0)
        mask = (lane // d) == sub

        qrow = q_ref[...]
        qexp = jnp.where(mask, jnp.broadcast_to(qrow, (h, hd)), 0.0)
        kb = k_ref[0]
        vb = v_ref[0]

        s = lax.dot_general(
            qexp, kb, (((1,), (1,)), ((), ())),
            preferred_element_type=jnp.float32,
        ) * scale
        m = jnp.max(s, axis=1, keepdims=True)
        p = jnp.exp(s - m)
        l = jnp.sum(p, axis=1, keepdims=True)

        g = lax.dot_general(
            p, vb, (((1,), (0,)), ((), ())),
            preferred_element_type=jnp.float32,
        )
        o_flat = jnp.sum(jnp.where(mask, g, 0.0), axis=0, keepdims=True)
        m_flat = jnp.sum(
            jnp.where(mask, jnp.broadcast_to(m, (h, hd)), 0.0),
            axis=0, keepdims=True)
        l_flat = jnp.sum(
            jnp.where(mask, jnp.broadcast_to(l, (h, hd)), 0.0),
            axis=0, keepdims=True)

        acc_ref[pl.ds(bi, 1), :] = o_flat
        acc_ref[pl.ds(b + bi, 1), :] = m_flat
        acc_ref[pl.ds(2 * b + bi, 1), :] = l_flat

        @pl.when(bi == b - 1)
        def _exchange_and_combine():
            rdma = pltpu.make_async_remote_copy(
                src_ref=acc_ref,
                dst_ref=recv_ref,
                send_sem=send_sem,
                recv_sem=recv_sem,
                device_id=partner,
                device_id_type=pl.DeviceIdType.MESH,
            )
            rdma.start()
            rdma.wait()

            oa = acc_ref[0:b, :]
            ma = acc_ref[b:2 * b, :]
            la = acc_ref[2 * b:3 * b, :]
            ob = recv_ref[0:b, :]
            mb = recv_ref[b:2 * b, :]
            lb = recv_ref[2 * b:3 * b, :]
            mn = jnp.maximum(ma, mb)
            alpha = jnp.exp(ma - mn)
            beta = jnp.exp(mb - mn)
            out_ref[...] = (alpha * oa + beta * ob) / (alpha * la + beta * lb)

    out = pl.pallas_call(
        body,
        grid=(b,),
        out_shape=jax.ShapeDtypeStruct((b, hd), jnp.float32),
        in_specs=[
            pl.BlockSpec((1, hd), lambda i: (i, 0)),
            pl.BlockSpec((1, k_len, hd), lambda i: (i, 0, 0)),
            pl.BlockSpec((1, k_len, hd), lambda i: (i, 0, 0)),
        ],
        out_specs=pl.BlockSpec((b, hd), lambda i: (0, 0)),
        scratch_shapes=[
            pltpu.VMEM((3 * b, hd), jnp.float32),
            pltpu.VMEM((3 * b, hd), jnp.float32),
            pltpu.SemaphoreType.DMA,
            pltpu.SemaphoreType.DMA,
        ],
        compiler_params=pltpu.CompilerParams(
            dimension_semantics=("arbitrary",),
            collective_id=0,
        ),
    )(Qf, Kf, Vf)

    return out.reshape(b, q_len, h, d)


# baseline (device time: 177849 ns/iter reference)
import jax
import jax.numpy as jnp
from jax import lax
from jax.experimental import pallas as pl
from jax.experimental.pallas import tpu as pltpu


def kernel(Q, K, V):
    b, q_len, h, d = Q.shape
    k_len = K.shape[1]
    hd = h * d
    scale = d ** -0.5

    Qf = Q.reshape(b, hd)
    Kf = K.reshape(b, k_len, hd)
    Vf = V.reshape(b, k_len, hd)

    def body(q_ref, k_ref, v_ref, out_ref, acc_ref, recv_ref, send_sem, recv_sem):
        bi = pl.program_id(0)
        mx = lax.axis_index("x")
        my = lax.axis_index("y")
        mz = lax.axis_index("z")
        partner = (mx, my, 1 - mz)

        @pl.when(bi == 0)
        def _entry_barrier():
            bar = pltpu.get_barrier_semaphore()
            pl.semaphore_signal(
                bar, inc=1, device_id=partner,
                device_id_type=pl.DeviceIdType.MESH,
            )
            pl.semaphore_wait(bar, 1)

        lane = lax.broadcasted_iota(jnp.int32, (h, hd), 1)
        sub = lax.broadcasted_iota(jnp.int32, (h, hd), 0)
        mask = (lane // d) == sub

        qrow = q_ref[pl.ds(bi, 1), :]
        qexp = jnp.where(mask, jnp.broadcast_to(qrow, (h, hd)), 0.0)
        kb = k_ref[0]
        vb = v_ref[0]

        s = lax.dot_general(
            qexp, kb, (((1,), (1,)), ((), ())),
            preferred_element_type=jnp.float32,
        ) * scale
        m = jnp.max(s, axis=1, keepdims=True)
        p = jnp.exp(s - m)
        l = jnp.sum(p, axis=1, keepdims=True)

        g = lax.dot_general(
            p, vb, (((1,), (0,)), ((), ())),
            preferred_element_type=jnp.float32,
        )
        o_flat = jnp.sum(jnp.where(mask, g, 0.0), axis=0, keepdims=True)
        m_flat = jnp.sum(
            jnp.where(mask, jnp.broadcast_to(m, (h, hd)), 0.0),
            axis=0, keepdims=True)
        l_flat = jnp.sum(
            jnp.where(mask, jnp.broadcast_to(l, (h, hd)), 0.0),
            axis=0, keepdims=True)

        acc_ref[pl.ds(bi, 1), :] = o_flat
        acc_ref[pl.ds(b + bi, 1), :] = m_flat
        acc_ref[pl.ds(2 * b + bi, 1), :] = l_flat

        @pl.when(bi == b - 1)
        def _exchange_and_combine():
            rdma = pltpu.make_async_remote_copy(
                src_ref=acc_ref,
                dst_ref=recv_ref,
                send_sem=send_sem,
                recv_sem=recv_sem,
                device_id=partner,
                device_id_type=pl.DeviceIdType.MESH,
            )
            rdma.start()
            rdma.wait()

            oa = acc_ref[0:b, :]
            ma = acc_ref[b:2 * b, :]
            la = acc_ref[2 * b:3 * b, :]
            ob = recv_ref[0:b, :]
            mb = recv_ref[b:2 * b, :]
            lb = recv_ref[2 * b:3 * b, :]
            mn = jnp.maximum(ma, mb)
            alpha = jnp.exp(ma - mn)
            beta = jnp.exp(mb - mn)
            out_ref[...] = (alpha * oa + beta * ob) / (alpha * la + beta * lb)

    out = pl.pallas_call(
        body,
        grid=(b,),
        out_shape=jax.ShapeDtypeStruct((b, hd), jnp.float32),
        in_specs=[
            pl.BlockSpec((b, hd), lambda i: (0, 0)),
            pl.BlockSpec((1, k_len, hd), lambda i: (i, 0, 0)),
            pl.BlockSpec((1, k_len, hd), lambda i: (i, 0, 0)),
        ],
        out_specs=pl.BlockSpec((b, hd), lambda i: (0, 0)),
        scratch_shapes=[
            pltpu.VMEM((3 * b, hd), jnp.float32),
            pltpu.VMEM((3 * b, hd), jnp.float32),
            pltpu.SemaphoreType.DMA,
            pltpu.SemaphoreType.DMA,
        ],
        compiler_params=pltpu.CompilerParams(
            dimension_semantics=("arbitrary",),
            collective_id=0,
        ),
    )(Qf, Kf, Vf)

    return out.reshape(b, q_len, h, d)
